# Initial kernel scaffold; baseline (speedup 1.0000x reference)
import jax
import jax.numpy as jnp
from jax.experimental import pallas as pl


def kernel(PLM_feat, LLM_feat, adj_t, Wd_l, Wd_r, bd, Wg_l, Wg_r, bg):
    raise NotImplementedError("write your pallas kernel here")


# speedup vs baseline: 6.0273x; 6.0273x over previous
import functools

import jax
import jax.numpy as jnp
from jax import lax
from jax.experimental import pallas as pl
from jax.experimental.pallas import tpu as pltpu
from jax.experimental.pallas import tpu_sc as plsc

N = 10000
D = 128
E = 320000
NC, NS = 2, 16
NH = N // NC
CHUNK = 128
CPT = -(-E // (NS * CHUNK))
E_PAD = NS * CPT * CHUNK
TR_BASE = 5120
TR = 1024
N_ACC = TR_BASE + TR
RPT = N_ACC // NS


def _make_sc_agg(with_deg: bool):
    mesh = plsc.VectorSubcoreMesh(
        core_axis_name="c", subcore_axis_name="s", num_cores=NC, num_subcores=NS
    )
    out_type = [jax.ShapeDtypeStruct((NC, N_ACC, D), jnp.float32)]
    if with_deg:
        out_type.append(jax.ShapeDtypeStruct((NC, N_ACC), jnp.float32))
    scratch = [
        pltpu.VMEM((CPT, CHUNK), jnp.int32),
        pltpu.VMEM((CPT, CHUNK), jnp.int32),
        pltpu.VMEM((CHUNK, D), jnp.float32),
        pltpu.VMEM((CHUNK, D), jnp.float32),
        pltpu.SemaphoreType.DMA,
        pltpu.SemaphoreType.DMA,
        pltpu.VMEM_SHARED((N_ACC, D), jnp.float32),
    ]
    if with_deg:
        scratch += [
            pltpu.VMEM((CHUNK,), jnp.float32),
            pltpu.VMEM_SHARED((N_ACC,), jnp.float32),
        ]

    @functools.partial(pl.kernel, mesh=mesh, out_type=out_type,
                       scratch_types=scratch)
    def k(*refs):
        if with_deg:
            (src_hbm, dst_hbm, x_hbm, zacc_hbm, zdeg_hbm, ones_hbm,
             acc_out, deg_out,
             src_v, dst_v, rows0, rows1, sem0, sem1, acc_sh,
             ones_v, deg_sh) = refs
        else:
            (src_hbm, dst_hbm, x_hbm, zacc_hbm,
             acc_out,
             src_v, dst_v, rows0, rows1, sem0, sem1, acc_sh) = refs

        c = lax.axis_index("c")
        s = lax.axis_index("s")
        base = s * RPT

        pltpu.sync_copy(zacc_hbm, acc_sh.at[pl.ds(base, RPT)])
        if with_deg:
            pltpu.sync_copy(zdeg_hbm, deg_sh.at[pl.ds(base, RPT)])
            pltpu.sync_copy(ones_hbm, ones_v)
        pltpu.sync_copy(src_hbm.at[s], src_v)
        pltpu.sync_copy(dst_hbm.at[c * NS + s], dst_v)
        plsc.subcore_barrier()

        d0 = pltpu.async_copy(x_hbm.at[src_v.at[0]], rows0, sem0)
        d0.wait()
        pltpu.sync_copy(rows0, acc_sh.at[dst_v.at[0]], add=True)
        if with_deg:
            pltpu.sync_copy(ones_v, deg_sh.at[dst_v.at[0]], add=True)

        def step(i, carry):
            j0 = 1 + 2 * i
            g0 = pltpu.async_copy(x_hbm.at[src_v.at[j0]], rows0, sem0)
            g1 = pltpu.async_copy(x_hbm.at[src_v.at[j0 + 1]], rows1, sem1)
            g0.wait()
            pltpu.sync_copy(rows0, acc_sh.at[dst_v.at[j0]], add=True)
            if with_deg:
                pltpu.sync_copy(ones_v, deg_sh.at[dst_v.at[j0]], add=True)
            g1.wait()
            pltpu.sync_copy(rows1, acc_sh.at[dst_v.at[j0 + 1]], add=True)
            if with_deg:
                pltpu.sync_copy(ones_v, deg_sh.at[dst_v.at[j0 + 1]], add=True)
            return carry

        lax.fori_loop(0, (CPT - 1) // 2, step, 0)

        plsc.subcore_barrier()
        pltpu.sync_copy(acc_sh.at[pl.ds(base, RPT)],
                        acc_out.at[c].at[pl.ds(base, RPT)])
        if with_deg:
            pltpu.sync_copy(deg_sh.at[pl.ds(base, RPT)],
                            deg_out.at[c].at[pl.ds(base, RPT)])

    return k


_sc_cache = {}


def _sc_agg_deg(*args):
    if True not in _sc_cache:
        _sc_cache[True] = _make_sc_agg(True)
    return _sc_cache[True](*args)


def _sc_agg(*args):
    if False not in _sc_cache:
        _sc_cache[False] = _make_sc_agg(False)
    return _sc_cache[False](*args)


R_TC = 1000


def _make_tc_combine(blend: bool):
    def body(pacc, pdeg, x, p, wl, wr, b, o):
        agg = pacc[0] / jnp.maximum(pdeg[0], 1.0)
        res = (jnp.dot(agg, wl[...], preferred_element_type=jnp.float32)
               + jnp.dot(x[...], wr[...], preferred_element_type=jnp.float32)
               + b[...])
        if blend:
            o[...] = 0.5 * res + 0.5 * p[...]
        else:
            o[...] = res

    def body_noblend(pacc, pdeg, x, wl, wr, b, o):
        body(pacc, pdeg, x, None, wl, wr, b, o)

    in_specs = [
        pl.BlockSpec((1, R_TC, D), lambda i: (i // 5, i % 5, 0)),
        pl.BlockSpec((1, R_TC, 1), lambda i: (i // 5, i % 5, 0)),
        pl.BlockSpec((R_TC, D), lambda i: (i, 0)),
    ]
    if blend:
        in_specs.append(pl.BlockSpec((R_TC, D), lambda i: (i, 0)))
    in_specs += [
        pl.BlockSpec((D, D), lambda i: (0, 0)),
        pl.BlockSpec((D, D), lambda i: (0, 0)),
        pl.BlockSpec((1, D), lambda i: (0, 0)),
    ]
    return pl.pallas_call(
        body if blend else body_noblend,
        grid=(N // R_TC,),
        in_specs=in_specs,
        out_specs=pl.BlockSpec((R_TC, D), lambda i: (i, 0)),
        out_shape=jax.ShapeDtypeStruct((N, D), jnp.float32),
    )


_tc_blend = _make_tc_combine(True)
_tc_plain = _make_tc_combine(False)


def kernel(PLM_feat, LLM_feat, adj_t, Wd_l, Wd_r, bd, Wg_l, Wg_r, bg):
    src = adj_t[0].astype(jnp.int32)
    dst = adj_t[1].astype(jnp.int32)
    npad = E_PAD - E
    arp = jnp.arange(npad, dtype=jnp.int32)
    src_p = jnp.concatenate([src, arp % N]).reshape(NS, CPT, CHUNK)
    dst_all = jnp.concatenate([dst, jnp.full((npad,), N, jnp.int32)])
    trash = TR_BASE + (jnp.arange(E_PAD, dtype=jnp.int32) % TR)
    dst_cs = []
    for cc in range(NC):
        lo = cc * NH
        inr = (dst_all >= lo) & (dst_all < lo + NH)
        dst_cs.append(jnp.where(inr, dst_all - lo, trash))
    dst_p = jnp.stack(dst_cs).reshape(NC * NS, CPT, CHUNK)

    zacc = jnp.zeros((RPT, D), jnp.float32)
    zdeg = jnp.zeros((RPT,), jnp.float32)
    ones = jnp.ones((CHUNK,), jnp.float32)

    pacc1, pdeg = _sc_agg_deg(src_p, dst_p, LLM_feat, zacc, zdeg, ones)
    pdeg3 = pdeg.reshape(NC, N_ACC, 1)
    feat = _tc_blend(pacc1, pdeg3, LLM_feat, PLM_feat, Wd_l, Wd_r,
                     bd.reshape(1, D))
    (pacc2,) = _sc_agg(src_p, dst_p, feat, zacc)
    h = _tc_plain(pacc2, pdeg3, feat, Wg_l, Wg_r, bg.reshape(1, D))
    return h


# speedup vs baseline: 8.3483x; 1.3851x over previous
import functools

import jax
import jax.numpy as jnp
from jax import lax
from jax.experimental import pallas as pl
from jax.experimental.pallas import tpu as pltpu
from jax.experimental.pallas import tpu_sc as plsc

N = 10000
D = 128
E = 320000
NC, NS = 2, 16
NH = N // NC
CHUNK = 128
CPT = -(-E // (NS * CHUNK))
E_PAD = NS * CPT * CHUNK
TR_BASE = 5120
TR = 1024
N_ACC = TR_BASE + TR
RPT = N_ACC // NS
NBUF = 2


def _make_sc_agg(with_deg: bool):
    mesh = plsc.VectorSubcoreMesh(
        core_axis_name="c", subcore_axis_name="s", num_cores=NC, num_subcores=NS
    )
    out_type = [jax.ShapeDtypeStruct((NC, N_ACC, D), jnp.float32)]
    if with_deg:
        out_type.append(jax.ShapeDtypeStruct((NC, N_ACC), jnp.float32))
    scratch = [
        pltpu.VMEM((CPT, CHUNK), jnp.int32),
        pltpu.VMEM((CPT, CHUNK), jnp.int32),
    ] + [pltpu.VMEM((CHUNK, D), jnp.float32) for _ in range(NBUF)] \
      + [pltpu.SemaphoreType.DMA for _ in range(NBUF)] + [
        pltpu.VMEM_SHARED((N_ACC, D), jnp.float32),
    ]
    if with_deg:
        scratch += [
            pltpu.VMEM((CHUNK,), jnp.float32),
            pltpu.VMEM_SHARED((N_ACC,), jnp.float32),
        ]

    @functools.partial(pl.kernel, mesh=mesh, out_type=out_type,
                       scratch_types=scratch)
    def k(*refs):
        if with_deg:
            (src_hbm, dst_hbm, x_hbm, zacc_hbm, zdeg_hbm, ones_hbm,
             acc_out, deg_out, src_v, dst_v) = refs[:10]
            bufs = refs[10:10 + NBUF]
            sems = refs[10 + NBUF:10 + 2 * NBUF]
            acc_sh, ones_v, deg_sh = refs[10 + 2 * NBUF:]
        else:
            (src_hbm, dst_hbm, x_hbm, zacc_hbm,
             acc_out, src_v, dst_v) = refs[:7]
            bufs = refs[7:7 + NBUF]
            sems = refs[7 + NBUF:7 + 2 * NBUF]
            (acc_sh,) = refs[7 + 2 * NBUF:]

        c = lax.axis_index("c")
        s = lax.axis_index("s")
        base = s * RPT

        pltpu.sync_copy(zacc_hbm, acc_sh.at[pl.ds(base, RPT)])
        if with_deg:
            pltpu.sync_copy(zdeg_hbm, deg_sh.at[pl.ds(base, RPT)])
            pltpu.sync_copy(ones_hbm, ones_v)
        pltpu.sync_copy(src_hbm.at[s], src_v)
        pltpu.sync_copy(dst_hbm.at[c * NS + s], dst_v)
        plsc.subcore_barrier()

        for b in range(NBUF):
            pltpu.async_copy(x_hbm.at[src_v.at[b]], bufs[b], sems[b])

        def step(i, carry):
            for b in range(NBUF):
                j = i * NBUF + b
                pltpu.make_async_copy(x_hbm.at[pl.ds(0, CHUNK)],
                                      bufs[b], sems[b]).wait()
                pltpu.sync_copy(bufs[b], acc_sh.at[dst_v.at[j]], add=True)
                if with_deg:
                    pltpu.sync_copy(ones_v, deg_sh.at[dst_v.at[j]], add=True)

                @pl.when(j + NBUF < CPT)
                def _():
                    pltpu.async_copy(x_hbm.at[src_v.at[j + NBUF]],
                                     bufs[b], sems[b])
            return carry

        lax.fori_loop(0, CPT // NBUF, step, 0)
        for j in range((CPT // NBUF) * NBUF, CPT):
            b = j % NBUF
            pltpu.make_async_copy(x_hbm.at[pl.ds(0, CHUNK)],
                                  bufs[b], sems[b]).wait()
            pltpu.sync_copy(bufs[b], acc_sh.at[dst_v.at[j]], add=True)
            if with_deg:
                pltpu.sync_copy(ones_v, deg_sh.at[dst_v.at[j]], add=True)

        plsc.subcore_barrier()
        pltpu.sync_copy(acc_sh.at[pl.ds(base, RPT)],
                        acc_out.at[c].at[pl.ds(base, RPT)])
        if with_deg:
            pltpu.sync_copy(deg_sh.at[pl.ds(base, RPT)],
                            deg_out.at[c].at[pl.ds(base, RPT)])

    return k


_sc_cache = {}


def _sc_agg_deg(*args):
    if True not in _sc_cache:
        _sc_cache[True] = _make_sc_agg(True)
    return _sc_cache[True](*args)


def _sc_agg(*args):
    if False not in _sc_cache:
        _sc_cache[False] = _make_sc_agg(False)
    return _sc_cache[False](*args)


R_TC = 1000


def _make_tc_combine(blend: bool):
    def body(pacc, pdeg, x, p, wl, wr, b, o):
        agg = pacc[0] / jnp.maximum(pdeg[0], 1.0)
        res = (jnp.dot(agg, wl[...], preferred_element_type=jnp.float32)
               + jnp.dot(x[...], wr[...], preferred_element_type=jnp.float32)
               + b[...])
        if blend:
            o[...] = 0.5 * res + 0.5 * p[...]
        else:
            o[...] = res

    def body_noblend(pacc, pdeg, x, wl, wr, b, o):
        body(pacc, pdeg, x, None, wl, wr, b, o)

    in_specs = [
        pl.BlockSpec((1, R_TC, D), lambda i: (i // 5, i % 5, 0)),
        pl.BlockSpec((1, R_TC, 1), lambda i: (i // 5, i % 5, 0)),
        pl.BlockSpec((R_TC, D), lambda i: (i, 0)),
    ]
    if blend:
        in_specs.append(pl.BlockSpec((R_TC, D), lambda i: (i, 0)))
    in_specs += [
        pl.BlockSpec((D, D), lambda i: (0, 0)),
        pl.BlockSpec((D, D), lambda i: (0, 0)),
        pl.BlockSpec((1, D), lambda i: (0, 0)),
    ]
    return pl.pallas_call(
        body if blend else body_noblend,
        grid=(N // R_TC,),
        in_specs=in_specs,
        out_specs=pl.BlockSpec((R_TC, D), lambda i: (i, 0)),
        out_shape=jax.ShapeDtypeStruct((N, D), jnp.float32),
    )


_tc_blend = _make_tc_combine(True)
_tc_plain = _make_tc_combine(False)


def kernel(PLM_feat, LLM_feat, adj_t, Wd_l, Wd_r, bd, Wg_l, Wg_r, bg):
    src = adj_t[0].astype(jnp.int32)
    dst = adj_t[1].astype(jnp.int32)
    npad = E_PAD - E
    arp = jnp.arange(npad, dtype=jnp.int32)
    src_p = jnp.concatenate([src, arp % N]).reshape(NS, CPT, CHUNK)
    dst_all = jnp.concatenate([dst, jnp.full((npad,), N, jnp.int32)])
    trash = TR_BASE + (jnp.arange(E_PAD, dtype=jnp.int32) % TR)
    dst_cs = []
    for cc in range(NC):
        lo = cc * NH
        inr = (dst_all >= lo) & (dst_all < lo + NH)
        dst_cs.append(jnp.where(inr, dst_all - lo, trash))
    dst_p = jnp.stack(dst_cs).reshape(NC * NS, CPT, CHUNK)

    zacc = jnp.zeros((RPT, D), jnp.float32)
    zdeg = jnp.zeros((RPT,), jnp.float32)
    ones = jnp.ones((CHUNK,), jnp.float32)

    pacc1, pdeg = _sc_agg_deg(src_p, dst_p, LLM_feat, zacc, zdeg, ones)
    pdeg3 = pdeg.reshape(NC, N_ACC, 1)
    feat = _tc_blend(pacc1, pdeg3, LLM_feat, PLM_feat, Wd_l, Wd_r,
                     bd.reshape(1, D))
    (pacc2,) = _sc_agg(src_p, dst_p, feat, zacc)
    h = _tc_plain(pacc2, pdeg3, feat, Wg_l, Wg_r, bg.reshape(1, D))
    return h


# speedup vs baseline: 12.5201x; 1.4997x over previous
import functools

import jax
import jax.numpy as jnp
from jax import lax
from jax.experimental import pallas as pl
from jax.experimental.pallas import tpu as pltpu
from jax.experimental.pallas import tpu_sc as plsc

N = 10000
D = 128
E = 320000
NC, NS = 2, 16
NH = N // NC
L = 16
CHUNK = 128
CPT = -(-E // (NS * CHUNK))
E_PAD = NS * CPT * CHUNK
CAP = CPT + 1
TR_BASE = 5120
TR = 1024
N_ACC = TR_BASE + TR
RPT = N_ACC // NS
NBUF = 2


def _make_sc_partition():
    mesh = plsc.VectorSubcoreMesh(
        core_axis_name="c", subcore_axis_name="s", num_cores=NC, num_subcores=NS
    )
    out_type = [
        jax.ShapeDtypeStruct((NC * NS, CAP, CHUNK), jnp.int32),
        jax.ShapeDtypeStruct((NC * NS, CAP, CHUNK), jnp.int32),
        jax.ShapeDtypeStruct((NC * NS, L), jnp.int32),
    ]
    scratch = [
        pltpu.VMEM((CPT, CHUNK), jnp.int32),
        pltpu.VMEM((CPT, CHUNK), jnp.int32),
        pltpu.VMEM((1, CAP * CHUNK), jnp.int32),
        pltpu.VMEM((1, CAP * CHUNK), jnp.int32),
        pltpu.VMEM((L,), jnp.int32),
    ]

    @functools.partial(pl.kernel, mesh=mesh, out_type=out_type,
                       scratch_types=scratch,
                       compiler_params=pltpu.CompilerParams(
                           needs_layout_passes=False))
    def k(src_hbm, dst_hbm, src_out, dst_out, cnt_out,
          src_v, dst_v, src_o, dst_o, cnt_v):
        c = lax.axis_index("c")
        s = lax.axis_index("s")
        w = c * NS + s
        lo = c * NH

        pltpu.sync_copy(src_hbm.at[s], src_v)
        pltpu.sync_copy(dst_hbm.at[s], dst_v)

        zero16 = jnp.zeros((L,), jnp.int32)

        @plsc.parallel_loop(0, CPT, carry=jnp.int32(0))
        def row(j, cur):
            for kk in range(CHUNK // L):
                sl = pl.ds(kk * L, L)
                dv = dst_v[j, sl]
                sv = src_v[j, sl]
                m = (dv >= lo) & (dv < lo + NH)
                mi = m.astype(jnp.int32)
                excl = plsc.cumsum(mi) - mi
                idx = jnp.where(m, cur + excl, CAP * CHUNK - 1)
                plsc.store_scatter(dst_o.at[0], [idx], dv - lo)
                plsc.store_scatter(src_o.at[0], [idx], sv)
                cur = cur + jnp.sum(mi)
            return cur

        cur = row

        pair = 2 * CHUNK
        target = jnp.maximum((cur + pair - 1) // pair, 1) * pair
        iota = jax.lax.iota(jnp.int32, L)

        def pad_body(t, cur):
            @pl.when(cur < target)
            def _():
                sp = (cur + iota) & 8191
                dp = TR_BASE + ((cur + iota) & (TR - 16))
                src_o[0, pl.ds(cur, L)] = sp
                dst_o[0, pl.ds(cur, L)] = dp
            return jnp.where(cur < target, cur + L, cur)

        cur = lax.fori_loop(0, 2 * CHUNK // L, pad_body, cur)

        cnt_v[...] = jnp.broadcast_to(target // CHUNK, (L,)).astype(jnp.int32)
        pltpu.sync_copy(cnt_v, cnt_out.at[w])
        pltpu.sync_copy(src_o.reshape(CAP, CHUNK), src_out.at[w])
        pltpu.sync_copy(dst_o.reshape(CAP, CHUNK), dst_out.at[w])

    return k


def _make_sc_agg(with_deg: bool):
    mesh = plsc.VectorSubcoreMesh(
        core_axis_name="c", subcore_axis_name="s", num_cores=NC, num_subcores=NS
    )
    out_type = [jax.ShapeDtypeStruct((NC, N_ACC, D), jnp.float32)]
    if with_deg:
        out_type.append(jax.ShapeDtypeStruct((NC, N_ACC), jnp.float32))
    scratch = [
        pltpu.VMEM((CAP, CHUNK), jnp.int32),
        pltpu.VMEM((CAP, CHUNK), jnp.int32),
        pltpu.VMEM((L,), jnp.int32),
    ] + [pltpu.VMEM((CHUNK, D), jnp.float32) for _ in range(NBUF)] \
      + [pltpu.SemaphoreType.DMA for _ in range(NBUF)] + [
        pltpu.VMEM_SHARED((N_ACC, D), jnp.float32),
    ]
    if with_deg:
        scratch += [
            pltpu.VMEM((CHUNK,), jnp.float32),
            pltpu.VMEM_SHARED((N_ACC,), jnp.float32),
        ]

    @functools.partial(pl.kernel, mesh=mesh, out_type=out_type,
                       scratch_types=scratch)
    def k(*refs):
        if with_deg:
            (src_hbm, dst_hbm, cnt_hbm, x_hbm, zacc_hbm, zdeg_hbm, ones_hbm,
             acc_out, deg_out, src_v, dst_v, cnt_v) = refs[:12]
            bufs = refs[12:12 + NBUF]
            sems = refs[12 + NBUF:12 + 2 * NBUF]
            acc_sh, ones_v, deg_sh = refs[12 + 2 * NBUF:]
        else:
            (src_hbm, dst_hbm, cnt_hbm, x_hbm, zacc_hbm,
             acc_out, src_v, dst_v, cnt_v) = refs[:9]
            bufs = refs[9:9 + NBUF]
            sems = refs[9 + NBUF:9 + 2 * NBUF]
            (acc_sh,) = refs[9 + 2 * NBUF:]

        c = lax.axis_index("c")
        s = lax.axis_index("s")
        w = c * NS + s
        base = s * RPT

        pltpu.sync_copy(zacc_hbm, acc_sh.at[pl.ds(base, RPT)])
        if with_deg:
            pltpu.sync_copy(zdeg_hbm, deg_sh.at[pl.ds(base, RPT)])
            pltpu.sync_copy(ones_hbm, ones_v)
        pltpu.sync_copy(src_hbm.at[w], src_v)
        pltpu.sync_copy(dst_hbm.at[w], dst_v)
        pltpu.sync_copy(cnt_hbm.at[w], cnt_v)
        nc = cnt_v[...][0]
        plsc.subcore_barrier()

        for b in range(NBUF):
            pltpu.async_copy(x_hbm.at[src_v.at[b]], bufs[b], sems[b])

        def step(i, carry):
            for b in range(NBUF):
                j = i * NBUF + b
                pltpu.make_async_copy(x_hbm.at[pl.ds(0, CHUNK)],
                                      bufs[b], sems[b]).wait()
                pltpu.sync_copy(bufs[b], acc_sh.at[dst_v.at[j]], add=True)
                if with_deg:
                    pltpu.sync_copy(ones_v, deg_sh.at[dst_v.at[j]], add=True)

                @pl.when(j + NBUF < nc)
                def _():
                    pltpu.async_copy(x_hbm.at[src_v.at[j + NBUF]],
                                     bufs[b], sems[b])
            return carry

        lax.fori_loop(0, nc // NBUF, step, 0)

        plsc.subcore_barrier()
        pltpu.sync_copy(acc_sh.at[pl.ds(base, RPT)],
                        acc_out.at[c].at[pl.ds(base, RPT)])
        if with_deg:
            pltpu.sync_copy(deg_sh.at[pl.ds(base, RPT)],
                            deg_out.at[c].at[pl.ds(base, RPT)])

    return k


_sc_cache = {}


def _sc_partition(*args):
    if "p" not in _sc_cache:
        _sc_cache["p"] = _make_sc_partition()
    return _sc_cache["p"](*args)


def _sc_agg_deg(*args):
    if True not in _sc_cache:
        _sc_cache[True] = _make_sc_agg(True)
    return _sc_cache[True](*args)


def _sc_agg(*args):
    if False not in _sc_cache:
        _sc_cache[False] = _make_sc_agg(False)
    return _sc_cache[False](*args)


R_TC = 1000


def _make_tc_combine(blend: bool):
    def body(pacc, pdeg, x, p, wl, wr, b, o):
        agg = pacc[0] / jnp.maximum(pdeg[0], 1.0)
        res = (jnp.dot(agg, wl[...], preferred_element_type=jnp.float32)
               + jnp.dot(x[...], wr[...], preferred_element_type=jnp.float32)
               + b[...])
        if blend:
            o[...] = 0.5 * res + 0.5 * p[...]
        else:
            o[...] = res

    def body_noblend(pacc, pdeg, x, wl, wr, b, o):
        body(pacc, pdeg, x, None, wl, wr, b, o)

    in_specs = [
        pl.BlockSpec((1, R_TC, D), lambda i: (i // 5, i % 5, 0)),
        pl.BlockSpec((1, R_TC, 1), lambda i: (i // 5, i % 5, 0)),
        pl.BlockSpec((R_TC, D), lambda i: (i, 0)),
    ]
    if blend:
        in_specs.append(pl.BlockSpec((R_TC, D), lambda i: (i, 0)))
    in_specs += [
        pl.BlockSpec((D, D), lambda i: (0, 0)),
        pl.BlockSpec((D, D), lambda i: (0, 0)),
        pl.BlockSpec((1, D), lambda i: (0, 0)),
    ]
    return pl.pallas_call(
        body if blend else body_noblend,
        grid=(N // R_TC,),
        in_specs=in_specs,
        out_specs=pl.BlockSpec((R_TC, D), lambda i: (i, 0)),
        out_shape=jax.ShapeDtypeStruct((N, D), jnp.float32),
    )


_tc_blend = _make_tc_combine(True)
_tc_plain = _make_tc_combine(False)


def kernel(PLM_feat, LLM_feat, adj_t, Wd_l, Wd_r, bd, Wg_l, Wg_r, bg):
    src = adj_t[0].astype(jnp.int32)
    dst = adj_t[1].astype(jnp.int32)
    npad = E_PAD - E
    arp = jnp.arange(npad, dtype=jnp.int32)
    src_p = jnp.concatenate([src, arp % N]).reshape(NS, CPT, CHUNK)
    dst_p = jnp.concatenate([dst, jnp.full((npad,), N, jnp.int32)])
    dst_p = dst_p.reshape(NS, CPT, CHUNK)

    zacc = jnp.zeros((RPT, D), jnp.float32)
    zdeg = jnp.zeros((RPT,), jnp.float32)
    ones = jnp.ones((CHUNK,), jnp.float32)

    src_r, dst_r, ncnk = _sc_partition(src_p, dst_p)
    pacc1, pdeg = _sc_agg_deg(src_r, dst_r, ncnk, LLM_feat, zacc, zdeg, ones)
    pdeg3 = pdeg.reshape(NC, N_ACC, 1)
    feat = _tc_blend(pacc1, pdeg3, LLM_feat, PLM_feat, Wd_l, Wd_r,
                     bd.reshape(1, D))
    (pacc2,) = _sc_agg(src_r, dst_r, ncnk, feat, zacc)
    h = _tc_plain(pacc2, pdeg3, feat, Wg_l, Wg_r, bg.reshape(1, D))
    return h


# speedup vs baseline: 12.9862x; 1.0372x over previous
import functools

import jax
import jax.numpy as jnp
from jax import lax
from jax.experimental import pallas as pl
from jax.experimental.pallas import tpu as pltpu
from jax.experimental.pallas import tpu_sc as plsc

N = 10000
D = 128
E = 320000
NC, NS = 2, 16
NH = N // NC
L = 16
CHUNK = 128
EPT = E // NS
CAP = 158
TR_BASE = 5120
TR = 1024
N_ACC = TR_BASE + TR
RPT = N_ACC // NS
NBUF = 2


def _make_sc_partition():
    mesh = plsc.VectorSubcoreMesh(
        core_axis_name="c", subcore_axis_name="s", num_cores=NC, num_subcores=NS
    )
    out_type = [
        jax.ShapeDtypeStruct((NC * NS, CAP, CHUNK), jnp.int32),
        jax.ShapeDtypeStruct((NC * NS, CAP, CHUNK), jnp.int32),
        jax.ShapeDtypeStruct((NC * NS, L), jnp.int32),
    ]
    scratch = [
        pltpu.VMEM((1, EPT), jnp.int32),
        pltpu.VMEM((1, EPT), jnp.int32),
        pltpu.VMEM((1, CAP * CHUNK), jnp.int32),
        pltpu.VMEM((1, CAP * CHUNK), jnp.int32),
        pltpu.VMEM((L,), jnp.int32),
    ]

    @functools.partial(pl.kernel, mesh=mesh, out_type=out_type,
                       scratch_types=scratch,
                       compiler_params=pltpu.CompilerParams(
                           needs_layout_passes=False))
    def k(src_hbm, dst_hbm, src_out, dst_out, cnt_out,
          src_v, dst_v, src_o, dst_o, cnt_v):
        c = lax.axis_index("c")
        s = lax.axis_index("s")
        w = c * NS + s
        lo = c * NH

        pltpu.sync_copy(src_hbm.at[s], src_v)
        pltpu.sync_copy(dst_hbm.at[s], dst_v)

        zero16 = jnp.zeros((L,), jnp.int32)

        @plsc.parallel_loop(0, EPT // L, carry=jnp.int32(0))
        def row(v, cur):
            if True:
                sl = pl.ds(v * L, L)
                dv = dst_v[0, sl]
                sv = src_v[0, sl]
                m = (dv >= lo) & (dv < lo + NH)
                mi = m.astype(jnp.int32)
                excl = plsc.cumsum(mi) - mi
                idx = jnp.where(m, cur + excl, CAP * CHUNK - 1)
                plsc.store_scatter(dst_o.at[0], [idx], dv - lo)
                plsc.store_scatter(src_o.at[0], [idx], sv)
                cur = cur + jnp.sum(mi)
            return cur

        cur = row

        pair = 2 * CHUNK
        target = jnp.maximum((cur + pair - 1) // pair, 1) * pair
        iota = jax.lax.iota(jnp.int32, L)

        def pad_body(t, cur):
            @pl.when(cur < target)
            def _():
                sp = (cur + iota) & 8191
                dp = TR_BASE + ((cur + iota) & (TR - 16))
                src_o[0, pl.ds(cur, L)] = sp
                dst_o[0, pl.ds(cur, L)] = dp
            return jnp.where(cur < target, cur + L, cur)

        cur = lax.fori_loop(0, 2 * CHUNK // L, pad_body, cur)

        cnt_v[...] = jnp.broadcast_to(target // CHUNK, (L,)).astype(jnp.int32)
        pltpu.sync_copy(cnt_v, cnt_out.at[w])
        pltpu.sync_copy(src_o.reshape(CAP, CHUNK), src_out.at[w])
        pltpu.sync_copy(dst_o.reshape(CAP, CHUNK), dst_out.at[w])

    return k


def _make_sc_agg(with_deg: bool):
    mesh = plsc.VectorSubcoreMesh(
        core_axis_name="c", subcore_axis_name="s", num_cores=NC, num_subcores=NS
    )
    out_type = [jax.ShapeDtypeStruct((NC, N_ACC, D), jnp.float32)]
    if with_deg:
        out_type.append(jax.ShapeDtypeStruct((NC, N_ACC), jnp.float32))
    scratch = [
        pltpu.VMEM((CAP, CHUNK), jnp.int32),
        pltpu.VMEM((CAP, CHUNK), jnp.int32),
        pltpu.VMEM((L,), jnp.int32),
    ] + [pltpu.VMEM((CHUNK, D), jnp.float32) for _ in range(NBUF)] \
      + [pltpu.SemaphoreType.DMA for _ in range(NBUF)] + [
        pltpu.VMEM_SHARED((N_ACC, D), jnp.float32),
    ]
    if with_deg:
        scratch += [
            pltpu.VMEM((CHUNK,), jnp.float32),
            pltpu.VMEM_SHARED((N_ACC,), jnp.float32),
        ]

    @functools.partial(pl.kernel, mesh=mesh, out_type=out_type,
                       scratch_types=scratch)
    def k(*refs):
        if with_deg:
            (src_hbm, dst_hbm, cnt_hbm, x_hbm, zacc_hbm, zdeg_hbm, ones_hbm,
             acc_out, deg_out, src_v, dst_v, cnt_v) = refs[:12]
            bufs = refs[12:12 + NBUF]
            sems = refs[12 + NBUF:12 + 2 * NBUF]
            acc_sh, ones_v, deg_sh = refs[12 + 2 * NBUF:]
        else:
            (src_hbm, dst_hbm, cnt_hbm, x_hbm, zacc_hbm,
             acc_out, src_v, dst_v, cnt_v) = refs[:9]
            bufs = refs[9:9 + NBUF]
            sems = refs[9 + NBUF:9 + 2 * NBUF]
            (acc_sh,) = refs[9 + 2 * NBUF:]

        c = lax.axis_index("c")
        s = lax.axis_index("s")
        w = c * NS + s
        base = s * RPT

        pltpu.sync_copy(zacc_hbm, acc_sh.at[pl.ds(base, RPT)])
        if with_deg:
            pltpu.sync_copy(zdeg_hbm, deg_sh.at[pl.ds(base, RPT)])
            pltpu.sync_copy(ones_hbm, ones_v)
        pltpu.sync_copy(src_hbm.at[w], src_v)
        pltpu.sync_copy(dst_hbm.at[w], dst_v)
        pltpu.sync_copy(cnt_hbm.at[w], cnt_v)
        nc = cnt_v[...][0]
        plsc.subcore_barrier()

        for b in range(NBUF):
            pltpu.async_copy(x_hbm.at[src_v.at[b]], bufs[b], sems[b])

        def step(i, carry):
            for b in range(NBUF):
                j = i * NBUF + b
                pltpu.make_async_copy(x_hbm.at[pl.ds(0, CHUNK)],
                                      bufs[b], sems[b]).wait()
                pltpu.sync_copy(bufs[b], acc_sh.at[dst_v.at[j]], add=True)
                if with_deg:
                    pltpu.sync_copy(ones_v, deg_sh.at[dst_v.at[j]], add=True)

                @pl.when(j + NBUF < nc)
                def _():
                    pltpu.async_copy(x_hbm.at[src_v.at[j + NBUF]],
                                     bufs[b], sems[b])
            return carry

        lax.fori_loop(0, nc // NBUF, step, 0)

        plsc.subcore_barrier()
        pltpu.sync_copy(acc_sh.at[pl.ds(base, RPT)],
                        acc_out.at[c].at[pl.ds(base, RPT)])
        if with_deg:
            pltpu.sync_copy(deg_sh.at[pl.ds(base, RPT)],
                            deg_out.at[c].at[pl.ds(base, RPT)])

    return k


_sc_cache = {}


def _sc_partition(*args):
    if "p" not in _sc_cache:
        _sc_cache["p"] = _make_sc_partition()
    return _sc_cache["p"](*args)


def _sc_agg_deg(*args):
    if True not in _sc_cache:
        _sc_cache[True] = _make_sc_agg(True)
    return _sc_cache[True](*args)


def _sc_agg(*args):
    if False not in _sc_cache:
        _sc_cache[False] = _make_sc_agg(False)
    return _sc_cache[False](*args)


R_TC = 1000


def _make_tc_combine(blend: bool):
    def body(pacc, pdeg, x, p, wl, wr, b, o):
        agg = pacc[0] / jnp.maximum(pdeg[0], 1.0)
        res = (jnp.dot(agg, wl[...], preferred_element_type=jnp.float32)
               + jnp.dot(x[...], wr[...], preferred_element_type=jnp.float32)
               + b[...])
        if blend:
            o[...] = 0.5 * res + 0.5 * p[...]
        else:
            o[...] = res

    def body_noblend(pacc, pdeg, x, wl, wr, b, o):
        body(pacc, pdeg, x, None, wl, wr, b, o)

    in_specs = [
        pl.BlockSpec((1, R_TC, D), lambda i: (i // 5, i % 5, 0)),
        pl.BlockSpec((1, R_TC, 1), lambda i: (i // 5, i % 5, 0)),
        pl.BlockSpec((R_TC, D), lambda i: (i, 0)),
    ]
    if blend:
        in_specs.append(pl.BlockSpec((R_TC, D), lambda i: (i, 0)))
    in_specs += [
        pl.BlockSpec((D, D), lambda i: (0, 0)),
        pl.BlockSpec((D, D), lambda i: (0, 0)),
        pl.BlockSpec((1, D), lambda i: (0, 0)),
    ]
    return pl.pallas_call(
        body if blend else body_noblend,
        grid=(N // R_TC,),
        in_specs=in_specs,
        out_specs=pl.BlockSpec((R_TC, D), lambda i: (i, 0)),
        out_shape=jax.ShapeDtypeStruct((N, D), jnp.float32),
    )


_tc_blend = _make_tc_combine(True)
_tc_plain = _make_tc_combine(False)


def kernel(PLM_feat, LLM_feat, adj_t, Wd_l, Wd_r, bd, Wg_l, Wg_r, bg):
    src_p = adj_t[0].astype(jnp.int32).reshape(NS, 1, EPT)
    dst_p = adj_t[1].astype(jnp.int32).reshape(NS, 1, EPT)

    zacc = jnp.zeros((RPT, D), jnp.float32)
    zdeg = jnp.zeros((RPT,), jnp.float32)
    ones = jnp.ones((CHUNK,), jnp.float32)

    src_r, dst_r, ncnk = _sc_partition(src_p, dst_p)
    pacc1, pdeg = _sc_agg_deg(src_r, dst_r, ncnk, LLM_feat, zacc, zdeg, ones)
    pdeg3 = pdeg.reshape(NC, N_ACC, 1)
    feat = _tc_blend(pacc1, pdeg3, LLM_feat, PLM_feat, Wd_l, Wd_r,
                     bd.reshape(1, D))
    (pacc2,) = _sc_agg(src_r, dst_r, ncnk, feat, zacc)
    h = _tc_plain(pacc2, pdeg3, feat, Wg_l, Wg_r, bg.reshape(1, D))
    return h


# speedup vs baseline: 12.9898x; 1.0003x over previous
import functools

import jax
import jax.numpy as jnp
from jax import lax
from jax.experimental import pallas as pl
from jax.experimental.pallas import tpu as pltpu
from jax.experimental.pallas import tpu_sc as plsc

N = 10000
D = 128
E = 320000
NC, NS = 2, 16
NH = N // NC
L = 16
CHUNK = 128
EPT = E // NS
CAP = 158
TR_BASE = 5120
TR = 1024
N_ACC = TR_BASE + TR
RPT = N_ACC // NS
NBUF = 2


def _make_sc_partition():
    mesh = plsc.VectorSubcoreMesh(
        core_axis_name="c", subcore_axis_name="s", num_cores=NC, num_subcores=NS
    )
    out_type = [
        jax.ShapeDtypeStruct((NC * NS, CAP, CHUNK), jnp.int32),
        jax.ShapeDtypeStruct((NC * NS, CAP, CHUNK), jnp.int32),
        jax.ShapeDtypeStruct((NC * NS, L), jnp.int32),
    ]
    scratch = [
        pltpu.VMEM((1, EPT), jnp.int32),
        pltpu.VMEM((1, EPT), jnp.int32),
        pltpu.VMEM((1, CAP * CHUNK), jnp.int32),
        pltpu.VMEM((1, CAP * CHUNK), jnp.int32),
        pltpu.VMEM((L,), jnp.int32),
    ]

    @functools.partial(pl.kernel, mesh=mesh, out_type=out_type,
                       scratch_types=scratch,
                       compiler_params=pltpu.CompilerParams(
                           needs_layout_passes=False))
    def k(src_hbm, dst_hbm, src_out, dst_out, cnt_out,
          src_v, dst_v, src_o, dst_o, cnt_v):
        c = lax.axis_index("c")
        s = lax.axis_index("s")
        w = c * NS + s
        lo = c * NH

        pltpu.sync_copy(src_hbm.at[s], src_v)
        pltpu.sync_copy(dst_hbm.at[s], dst_v)

        @plsc.parallel_loop(0, EPT // L, carry=jnp.int32(0))
        def row(v, cur):
            sl = pl.ds(v * L, L)
            dv = dst_v[0, sl]
            sv = src_v[0, sl]
            m = (dv >= lo) & (dv < lo + NH)
            mi = m.astype(jnp.int32)
            excl = plsc.cumsum(mi) - mi
            idx = jnp.where(m, cur + excl, CAP * CHUNK - 1)
            plsc.store_scatter(dst_o.at[0], [idx], dv - lo)
            plsc.store_scatter(src_o.at[0], [idx], sv)
            return cur + jnp.sum(mi)

        cur = row

        pair = 2 * CHUNK
        target = jnp.maximum((cur + pair - 1) // pair, 1) * pair
        iota = jax.lax.iota(jnp.int32, L)

        def pad_body(t, cur):
            @pl.when(cur < target)
            def _():
                sp = (cur + iota) & 8191
                dp = TR_BASE + ((cur + iota) & (TR - 16))
                src_o[0, pl.ds(cur, L)] = sp
                dst_o[0, pl.ds(cur, L)] = dp
            return jnp.where(cur < target, cur + L, cur)

        cur = lax.fori_loop(0, 2 * CHUNK // L, pad_body, cur)

        cnt_v[...] = jnp.broadcast_to(target // CHUNK, (L,)).astype(jnp.int32)
        pltpu.sync_copy(cnt_v, cnt_out.at[w])
        pltpu.sync_copy(src_o.reshape(CAP, CHUNK), src_out.at[w])
        pltpu.sync_copy(dst_o.reshape(CAP, CHUNK), dst_out.at[w])

    return k


def _make_sc_agg(with_deg: bool):
    mesh = plsc.VectorSubcoreMesh(
        core_axis_name="c", subcore_axis_name="s", num_cores=NC, num_subcores=NS
    )
    out_type = [jax.ShapeDtypeStruct((NC, N_ACC, D), jnp.float32)]
    if with_deg:
        out_type.append(jax.ShapeDtypeStruct((NC, N_ACC), jnp.float32))
    scratch = [
        pltpu.VMEM((CAP, CHUNK), jnp.int32),
        pltpu.VMEM((CAP, CHUNK), jnp.int32),
        pltpu.VMEM((L,), jnp.int32),
    ] + [pltpu.VMEM((CHUNK, D), jnp.float32) for _ in range(NBUF)] \
      + [pltpu.SemaphoreType.DMA for _ in range(NBUF)] + [
        pltpu.VMEM_SHARED((N_ACC, D), jnp.float32),
    ]
    if with_deg:
        scratch += [
            pltpu.VMEM((CHUNK,), jnp.float32),
            pltpu.VMEM_SHARED((N_ACC,), jnp.float32),
        ]

    @functools.partial(pl.kernel, mesh=mesh, out_type=out_type,
                       scratch_types=scratch)
    def k(*refs):
        if with_deg:
            (src_hbm, dst_hbm, cnt_hbm, x_hbm, zacc_hbm, zdeg_hbm, ones_hbm,
             acc_out, deg_out, src_v, dst_v, cnt_v) = refs[:12]
            bufs = refs[12:12 + NBUF]
            sems = refs[12 + NBUF:12 + 2 * NBUF]
            acc_sh, ones_v, deg_sh = refs[12 + 2 * NBUF:]
        else:
            (src_hbm, dst_hbm, cnt_hbm, x_hbm, zacc_hbm,
             acc_out, src_v, dst_v, cnt_v) = refs[:9]
            bufs = refs[9:9 + NBUF]
            sems = refs[9 + NBUF:9 + 2 * NBUF]
            (acc_sh,) = refs[9 + 2 * NBUF:]

        c = lax.axis_index("c")
        s = lax.axis_index("s")
        w = c * NS + s
        base = s * RPT

        pltpu.sync_copy(zacc_hbm, acc_sh.at[pl.ds(base, RPT)])
        if with_deg:
            pltpu.sync_copy(zdeg_hbm, deg_sh.at[pl.ds(base, RPT)])
            pltpu.sync_copy(ones_hbm, ones_v)
        pltpu.sync_copy(src_hbm.at[w], src_v)
        pltpu.sync_copy(dst_hbm.at[w], dst_v)
        pltpu.sync_copy(cnt_hbm.at[w], cnt_v)
        nc = cnt_v[...][0]
        plsc.subcore_barrier()

        for b in range(NBUF):
            pltpu.async_copy(x_hbm.at[src_v.at[b]], bufs[b], sems[b])

        def step(i, carry):
            for b in range(NBUF):
                j = i * NBUF + b
                pltpu.make_async_copy(x_hbm.at[pl.ds(0, CHUNK)],
                                      bufs[b], sems[b]).wait()
                pltpu.sync_copy(bufs[b], acc_sh.at[dst_v.at[j]], add=True)
                if with_deg:
                    pltpu.sync_copy(ones_v, deg_sh.at[dst_v.at[j]], add=True)

                @pl.when(j + NBUF < nc)
                def _():
                    pltpu.async_copy(x_hbm.at[src_v.at[j + NBUF]],
                                     bufs[b], sems[b])
            return carry

        lax.fori_loop(0, nc // NBUF, step, 0)

        plsc.subcore_barrier()
        pltpu.sync_copy(acc_sh.at[pl.ds(base, RPT)],
                        acc_out.at[c].at[pl.ds(base, RPT)])
        if with_deg:
            pltpu.sync_copy(deg_sh.at[pl.ds(base, RPT)],
                            deg_out.at[c].at[pl.ds(base, RPT)])

    return k


_sc_cache = {}


def _sc_partition(*args):
    if "p" not in _sc_cache:
        _sc_cache["p"] = _make_sc_partition()
    return _sc_cache["p"](*args)


def _sc_agg_deg(*args):
    if True not in _sc_cache:
        _sc_cache[True] = _make_sc_agg(True)
    return _sc_cache[True](*args)


def _sc_agg(*args):
    if False not in _sc_cache:
        _sc_cache[False] = _make_sc_agg(False)
    return _sc_cache[False](*args)


R_TC = 1000


def _make_tc_combine(blend: bool):
    def body(pacc, pdeg, x, p, wl, wr, b, o):
        agg = pacc[0] / jnp.maximum(pdeg[0], 1.0)
        res = (jnp.dot(agg, wl[...], preferred_element_type=jnp.float32)
               + jnp.dot(x[...], wr[...], preferred_element_type=jnp.float32)
               + b[...])
        if blend:
            o[...] = 0.5 * res + 0.5 * p[...]
        else:
            o[...] = res

    def body_noblend(pacc, pdeg, x, wl, wr, b, o):
        body(pacc, pdeg, x, None, wl, wr, b, o)

    in_specs = [
        pl.BlockSpec((1, R_TC, D), lambda i: (i // 5, i % 5, 0)),
        pl.BlockSpec((1, R_TC, 1), lambda i: (i // 5, i % 5, 0)),
        pl.BlockSpec((R_TC, D), lambda i: (i, 0)),
    ]
    if blend:
        in_specs.append(pl.BlockSpec((R_TC, D), lambda i: (i, 0)))
    in_specs += [
        pl.BlockSpec((D, D), lambda i: (0, 0)),
        pl.BlockSpec((D, D), lambda i: (0, 0)),
        pl.BlockSpec((1, D), lambda i: (0, 0)),
    ]
    return pl.pallas_call(
        body if blend else body_noblend,
        grid=(N // R_TC,),
        in_specs=in_specs,
        out_specs=pl.BlockSpec((R_TC, D), lambda i: (i, 0)),
        out_shape=jax.ShapeDtypeStruct((N, D), jnp.float32),
    )


_tc_blend = _make_tc_combine(True)
_tc_plain = _make_tc_combine(False)


def kernel(PLM_feat, LLM_feat, adj_t, Wd_l, Wd_r, bd, Wg_l, Wg_r, bg):
    src_p = adj_t[0].astype(jnp.int32).reshape(NS, 1, EPT)
    dst_p = adj_t[1].astype(jnp.int32).reshape(NS, 1, EPT)

    zacc = jnp.zeros((RPT, D), jnp.float32)
    zdeg = jnp.zeros((RPT,), jnp.float32)
    ones = jnp.ones((CHUNK,), jnp.float32)

    src_r, dst_r, ncnk = _sc_partition(src_p, dst_p)
    pacc1, pdeg = _sc_agg_deg(src_r, dst_r, ncnk, LLM_feat, zacc, zdeg, ones)
    pdeg3 = pdeg.reshape(NC, N_ACC, 1)
    feat = _tc_blend(pacc1, pdeg3, LLM_feat, PLM_feat, Wd_l, Wd_r,
                     bd.reshape(1, D))
    (pacc2,) = _sc_agg(src_r, dst_r, ncnk, feat, zacc)
    h = _tc_plain(pacc2, pdeg3, feat, Wg_l, Wg_r, bg.reshape(1, D))
    return h
